# baseline (speedup 1.0000x reference)
import functools

import jax
import jax.numpy as jnp
from jax import lax
from jax.experimental import pallas as pl
from jax.experimental.pallas import tpu as pltpu


def _round_up(x, m):
    return ((x + m - 1) // m) * m


def _pick_lane_tile(n, cap):
    best = None
    t = 128
    while t <= min(n, cap):
        if n % t == 0:
            best = t
        t += 128
    return best if best is not None else n


def _fused_kernel(x_ref, wbig_ref, cbig_ref, fw_ref, fb_ref,
                  u_ref, fnw1_ref, fnb1_ref, fnw2_ref, fnb2_ref,
                  a_ref, w1_ref, b1_ref, w2_ref, b2_ref, w3_ref, b3_ref,
                  r_ref, w1a_ref, w1b_ref, cb1_ref, cw2_ref, cb2_ref,
                  o_ref, feat_sc, fg_sc,
                  *, n_img_steps, n1, num_graphs, node_size, conv_mid, inv_hw):
    s = pl.program_id(0)

    @pl.when(s == 0)
    def _init():
        feat_sc[...] = jnp.zeros_like(feat_sc)

    @pl.when(s < n_img_steps)
    def _backbone():
        x = x_ref[0]
        h = jnp.dot(wbig_ref[...], x, preferred_element_type=jnp.float32)
        h = jnp.maximum(h + cbig_ref[...], 0.0)
        ps = jnp.sum(h, axis=1)
        pooled = jnp.sum(ps.reshape(2, conv_mid, 8), axis=2) * inv_hw
        feat = jnp.dot(pooled.astype(jnp.bfloat16), fw_ref[...],
                       preferred_element_type=jnp.float32) + fb_ref[...]
        featb = jnp.maximum(feat, 0.0).astype(jnp.bfloat16)
        rows_i = lax.broadcasted_iota(jnp.int32, (feat_sc.shape[0], 1), 0)
        m0 = (rows_i == 2 * s).astype(jnp.bfloat16)
        m1 = (rows_i == 2 * s + 1).astype(jnp.bfloat16)
        feat_sc[...] += m0 * featb[0:1, :] + m1 * featb[1:2, :]

    @pl.when(s == n_img_steps)
    def _gcn():
        GN = num_graphs * node_size
        u = u_ref[...].astype(jnp.bfloat16)
        t = jnp.dot(u, fnw1_ref[...],
                    preferred_element_type=jnp.float32) + fnb1_ref[...]
        t = jnp.maximum(t, 0.0).astype(jnp.bfloat16)
        h0 = jnp.dot(t, fnw2_ref[...],
                     preferred_element_type=jnp.float32) + fnb2_ref[...]

        A = a_ref[...]
        h0b = h0.astype(jnp.bfloat16)
        tiled = jnp.concatenate([h0b] * num_graphs, axis=1)
        row_g = lax.broadcasted_iota(jnp.int32, (GN, GN), 0) // node_size
        col_g = lax.broadcasted_iota(jnp.int32, (GN, GN), 1) // node_size
        ht = jnp.where(row_g == col_g, tiled, jnp.bfloat16(0))
        z = jnp.dot(ht, w1_ref[...], preferred_element_type=jnp.float32)
        z = jnp.dot(A, z.astype(jnp.bfloat16),
                    preferred_element_type=jnp.float32) + b1_ref[...]
        h1 = jnp.maximum(z, 0.0).astype(jnp.bfloat16)

        z = jnp.dot(h1, w2_ref[...], preferred_element_type=jnp.float32)
        z = jnp.dot(A, z.astype(jnp.bfloat16),
                    preferred_element_type=jnp.float32) + b2_ref[...]
        h2 = jnp.maximum(z, 0.0).astype(jnp.bfloat16)

        z = jnp.dot(h2, w3_ref[...], preferred_element_type=jnp.float32)
        z = jnp.dot(A, z.astype(jnp.bfloat16),
                    preferred_element_type=jnp.float32) + b3_ref[...]
        fg = jnp.dot(r_ref[...], z.astype(jnp.bfloat16),
                     preferred_element_type=jnp.float32)
        fg_sc[...] = jnp.maximum(fg, 0.0).astype(jnp.bfloat16)

    @pl.when(s > n_img_steps)
    def _classifier():
        j = s - (n_img_steps + 1)

        @pl.when(j == 0)
        def _():
            o_ref[...] = jnp.zeros_like(o_ref)

        h = jnp.dot(feat_sc[...], w1a_ref[...],
                    preferred_element_type=jnp.float32)
        h = h + jnp.dot(fg_sc[...], w1b_ref[...],
                        preferred_element_type=jnp.float32)
        h = jnp.maximum(h + cb1_ref[...], 0.0).astype(jnp.bfloat16)
        o_ref[...] += jnp.dot(h, cw2_ref[...],
                              preferred_element_type=jnp.float32)

        @pl.when(j == n1 - 1)
        def _():
            o_ref[...] = o_ref[...] + cb2_ref[...]


def _build_norm_adj(node_num):
    idx = jnp.arange(node_num)
    A = jnp.zeros((node_num, node_num), jnp.float32)
    A = A.at[idx, (idx + 1) % node_num].set(1.0)
    A = A.at[(idx + 1) % node_num, idx].set(1.0)
    A = A + jnp.eye(node_num, dtype=jnp.float32)
    dinv = 1.0 / jnp.sqrt(A.sum(axis=1))
    return A * dinv[:, None] * dinv[None, :]


def kernel(batch_img, batch_u, conv_wt, conv_b, bk_fc_w, bk_fc_b,
           fn_w1, fn_b1, fn_w2, fn_b2, gcn_w1, gcn_b1, gcn_w2, gcn_b2,
           gcn_w3, gcn_b3, fc_w1a, fc_b1, fc_w1b, fc_w2, fc_b2):
    B, C, H, W = batch_img.shape
    HW = H * W
    G, node_num, Fin = batch_u.shape
    GN = G * node_num
    conv_mid = conv_wt.shape[0]
    feat_dim = bk_fc_w.shape[1]
    H256 = fn_w1.shape[1]
    ns = fn_w2.shape[1]
    H1 = gcn_w1.shape[1]
    H2 = gcn_w2.shape[1]
    Fo = gcn_w3.shape[1]
    F1 = fc_w1a.shape[0]
    F2 = fc_w1b.shape[0]
    HC = fc_w1a.shape[1]
    num_classes = fc_w2.shape[1]

    ipp = 2
    rows = 24 * ipp
    lanes = C * HW // 24
    n_img_steps = B // ipp
    x48 = batch_img.reshape(n_img_steps, rows, lanes).astype(jnp.bfloat16)
    w_big = jnp.kron(jnp.eye(ipp, dtype=conv_wt.dtype),
                     jnp.kron(conv_wt, jnp.eye(8, dtype=conv_wt.dtype)))
    cb_big = jnp.kron(jnp.ones((ipp, 1), conv_b.dtype),
                      jnp.kron(conv_b, jnp.ones((8, 1), conv_b.dtype)))
    fw_bf = bk_fc_w.astype(jnp.bfloat16)

    A_hat = _build_norm_adj(node_num)
    A_bd = jnp.kron(jnp.eye(G, dtype=jnp.float32), A_hat).astype(jnp.bfloat16)
    R = jnp.kron(jnp.eye(G, dtype=jnp.float32),
                 jnp.full((1, node_num), 1.0 / node_num, jnp.float32)
                 ).astype(jnp.bfloat16)
    u2d = batch_u.reshape(GN, Fin)

    t_n1 = _pick_lane_tile(HC, 512)
    n1 = HC // t_n1
    Np = _round_up(max(num_classes, 128), 128)
    w2p = jnp.pad(fc_w2, ((0, 0), (0, Np - num_classes)))
    b2p = jnp.pad(fc_b2, ((0, 0), (0, Np - num_classes)))

    n_steps = n_img_steps + 1 + n1
    gi = n_img_steps

    def img_idx(sv):
        return (jnp.minimum(sv, n_img_steps - 1), 0, 0)

    def cls_idx_col(sv):
        return (0, jnp.clip(sv - (gi + 1), 0, n1 - 1))

    def cls_idx_row(sv):
        return (jnp.clip(sv - (gi + 1), 0, n1 - 1), 0)

    const2 = lambda sv: (0, 0)
    const3 = lambda sv: (0, 0, 0)

    flops = (2 * B * HW * C * conv_mid + 2 * B * conv_mid * feat_dim
             + 2 * GN * (Fin * H256 + H256 * ns) + 2 * GN * GN * H1
             + 2 * GN * GN * (H1 + H2 + Fo) + 2 * GN * (H1 * H2 + H2 * Fo)
             + 2 * G * GN * Fo + 2 * B * (F1 * HC + F2 * HC + HC * Np))
    bytes_acc = (B * C * HW * 4 + conv_mid * feat_dim * 2
                 + GN * Fin * 4 + GN * GN * 2 + GN * (H1 + H2) * 2
                 + H2 * Fo * 2 + (F1 + F2) * HC * 2 + HC * Np * 2
                 + B * Np * 4)

    out = pl.pallas_call(
        functools.partial(_fused_kernel, n_img_steps=n_img_steps, n1=n1,
                          num_graphs=G, node_size=node_num,
                          conv_mid=conv_mid, inv_hw=1.0 / float(HW)),
        out_shape=jax.ShapeDtypeStruct((B, Np), jnp.float32),
        grid_spec=pltpu.PrefetchScalarGridSpec(
            num_scalar_prefetch=0,
            grid=(n_steps,),
            in_specs=[
                pl.BlockSpec((1, rows, lanes), img_idx),
                pl.BlockSpec((ipp * 8 * conv_mid, rows), const2),
                pl.BlockSpec((ipp * 8 * conv_mid, 1), const2),
                pl.BlockSpec((conv_mid, feat_dim), const2),
                pl.BlockSpec((1, feat_dim), const2),
                pl.BlockSpec((GN, Fin), const2),
                pl.BlockSpec((Fin, H256), const2),
                pl.BlockSpec((1, H256), const2),
                pl.BlockSpec((H256, ns), const2),
                pl.BlockSpec((1, ns), const2),
                pl.BlockSpec((GN, GN), const2),
                pl.BlockSpec((GN, H1), const2),
                pl.BlockSpec((1, H1), const2),
                pl.BlockSpec((H1, H2), const2),
                pl.BlockSpec((1, H2), const2),
                pl.BlockSpec((H2, Fo), const2),
                pl.BlockSpec((1, Fo), const2),
                pl.BlockSpec((G, GN), const2),
                pl.BlockSpec((F1, t_n1), cls_idx_col),
                pl.BlockSpec((F2, t_n1), cls_idx_col),
                pl.BlockSpec((1, t_n1), cls_idx_col),
                pl.BlockSpec((t_n1, Np), cls_idx_row),
                pl.BlockSpec((1, Np), const2),
            ],
            out_specs=pl.BlockSpec((B, Np), const2),
            scratch_shapes=[pltpu.VMEM((B, feat_dim), jnp.bfloat16),
                            pltpu.VMEM((G, Fo), jnp.bfloat16)],
        ),
        compiler_params=pltpu.CompilerParams(
            dimension_semantics=("arbitrary",),
            vmem_limit_bytes=100 * 1024 * 1024),
        cost_estimate=pl.CostEstimate(flops=flops, transcendentals=0,
                                      bytes_accessed=bytes_acc),
    )(x48, w_big, cb_big, fw_bf, bk_fc_b,
      u2d, fn_w1, fn_b1, fn_w2, fn_b2,
      A_bd, gcn_w1, gcn_b1, gcn_w2, gcn_b2, gcn_w3, gcn_b3, R,
      fc_w1a, fc_w1b, fc_b1, w2p, b2p)
    return out[:, :num_classes]


# speedup vs baseline: 1.0798x; 1.0798x over previous
import functools

import jax
import jax.numpy as jnp
from jax import lax
from jax.experimental import pallas as pl
from jax.experimental.pallas import tpu as pltpu


def _round_up(x, m):
    return ((x + m - 1) // m) * m


def _fused_kernel(x_ref,
                  wbig_ref, cbig_ref, fw_ref, fb_ref,
                  u_ref, fnw1_ref, fnb1_ref, fnw2_ref, fnb2_ref,
                  a_ref, w1_ref, b1_ref, w2_ref, b2_ref, w3_ref, b3_ref,
                  r_ref, w1a_ref, w1b_ref, cb1_ref, cw2_ref, cb2_ref,
                  o_ref,
                  feat_sc, fg_sc, bufs, sems,
                  *, n_img_steps, num_graphs, node_size, conv_mid, inv_hw):
    s = pl.program_id(0)
    gi = n_img_steps

    hbm_refs = (wbig_ref, cbig_ref, fw_ref, fb_ref,
                u_ref, fnw1_ref, fnb1_ref, fnw2_ref, fnb2_ref,
                a_ref, w1_ref, b1_ref, w2_ref, b2_ref, w3_ref, b3_ref,
                r_ref, w1a_ref, w1b_ref, cb1_ref, cw2_ref, cb2_ref)
    n_bk = 4

    @pl.when(s == 0)
    def _prologue():
        feat_sc[...] = jnp.zeros_like(feat_sc)
        for i, ref in enumerate(hbm_refs):
            pltpu.make_async_copy(ref, bufs[i], sems.at[i]).start()
        for i in range(n_bk):
            pltpu.make_async_copy(hbm_refs[i], bufs[i], sems.at[i]).wait()

    @pl.when(s == gi)
    def _wait_tail():
        for i in range(n_bk, len(hbm_refs)):
            pltpu.make_async_copy(hbm_refs[i], bufs[i], sems.at[i]).wait()

    (wbig, cbig, fw, fb, u_b, fnw1, fnb1, fnw2, fnb2,
     a_b, w1_b, b1_b, w2_b, b2_b, w3_b, b3_b, r_b,
     w1a, w1b, cb1, cw2, cb2) = bufs

    @pl.when(s < n_img_steps)
    def _backbone():
        x = x_ref[0].astype(jnp.bfloat16)
        h = jnp.dot(wbig[...], x, preferred_element_type=jnp.float32)
        h = jnp.maximum(h + cbig[...], 0.0)
        ps = jnp.sum(h, axis=1)
        pooled = jnp.sum(ps.reshape(2, conv_mid, 8), axis=2) * inv_hw
        feat = jnp.dot(pooled.astype(jnp.bfloat16), fw[...],
                       preferred_element_type=jnp.float32) + fb[...]
        featb = jnp.maximum(feat, 0.0).astype(jnp.bfloat16)
        rows_i = lax.broadcasted_iota(jnp.int32, (feat_sc.shape[0], 1), 0)
        m0 = (rows_i == 2 * s).astype(jnp.bfloat16)
        m1 = (rows_i == 2 * s + 1).astype(jnp.bfloat16)
        feat_sc[...] += m0 * featb[0:1, :] + m1 * featb[1:2, :]

    @pl.when(s == gi)
    def _gcn():
        GN = num_graphs * node_size
        u = u_b[...].astype(jnp.bfloat16)
        t = jnp.dot(u, fnw1[...],
                    preferred_element_type=jnp.float32) + fnb1[...]
        t = jnp.maximum(t, 0.0).astype(jnp.bfloat16)
        h0 = jnp.dot(t, fnw2[...],
                     preferred_element_type=jnp.float32) + fnb2[...]

        A = a_b[...]
        h0b = h0.astype(jnp.bfloat16)
        tiled = jnp.concatenate([h0b] * num_graphs, axis=1)
        row_g = lax.broadcasted_iota(jnp.int32, (GN, GN), 0) // node_size
        col_g = lax.broadcasted_iota(jnp.int32, (GN, GN), 1) // node_size
        ht = jnp.where(row_g == col_g, tiled, jnp.bfloat16(0))
        z = jnp.dot(ht, w1_b[...], preferred_element_type=jnp.float32)
        z = jnp.dot(A, z.astype(jnp.bfloat16),
                    preferred_element_type=jnp.float32) + b1_b[...]
        h1 = jnp.maximum(z, 0.0).astype(jnp.bfloat16)

        z = jnp.dot(h1, w2_b[...], preferred_element_type=jnp.float32)
        z = jnp.dot(A, z.astype(jnp.bfloat16),
                    preferred_element_type=jnp.float32) + b2_b[...]
        h2 = jnp.maximum(z, 0.0).astype(jnp.bfloat16)

        z = jnp.dot(h2, w3_b[...], preferred_element_type=jnp.float32)
        z = jnp.dot(A, z.astype(jnp.bfloat16),
                    preferred_element_type=jnp.float32) + b3_b[...]
        fg = jnp.dot(r_b[...], z.astype(jnp.bfloat16),
                     preferred_element_type=jnp.float32)
        fg_sc[...] = jnp.maximum(fg, 0.0).astype(jnp.bfloat16)

    @pl.when(s == gi + 1)
    def _classifier():
        h = jnp.dot(feat_sc[...], w1a[...],
                    preferred_element_type=jnp.float32)
        h = h + jnp.dot(fg_sc[...], w1b[...],
                        preferred_element_type=jnp.float32)
        h = jnp.maximum(h + cb1[...], 0.0).astype(jnp.bfloat16)
        o_ref[...] = jnp.dot(h, cw2[...],
                             preferred_element_type=jnp.float32) + cb2[...]


def _build_norm_adj(node_num):
    idx = jnp.arange(node_num)
    A = jnp.zeros((node_num, node_num), jnp.float32)
    A = A.at[idx, (idx + 1) % node_num].set(1.0)
    A = A.at[(idx + 1) % node_num, idx].set(1.0)
    A = A + jnp.eye(node_num, dtype=jnp.float32)
    dinv = 1.0 / jnp.sqrt(A.sum(axis=1))
    return A * dinv[:, None] * dinv[None, :]


def kernel(batch_img, batch_u, conv_wt, conv_b, bk_fc_w, bk_fc_b,
           fn_w1, fn_b1, fn_w2, fn_b2, gcn_w1, gcn_b1, gcn_w2, gcn_b2,
           gcn_w3, gcn_b3, fc_w1a, fc_b1, fc_w1b, fc_w2, fc_b2):
    B, C, H, W = batch_img.shape
    HW = H * W
    G, node_num, Fin = batch_u.shape
    GN = G * node_num
    conv_mid = conv_wt.shape[0]
    feat_dim = bk_fc_w.shape[1]
    H256 = fn_w1.shape[1]
    ns = fn_w2.shape[1]
    H1 = gcn_w1.shape[1]
    H2 = gcn_w2.shape[1]
    Fo = gcn_w3.shape[1]
    F1 = fc_w1a.shape[0]
    F2 = fc_w1b.shape[0]
    HC = fc_w1a.shape[1]
    num_classes = fc_w2.shape[1]

    ipp = 2
    rows = 24 * ipp
    lanes = C * HW // 24
    n_img_steps = B // ipp
    x48 = batch_img.reshape(n_img_steps, rows, lanes)
    w_big = jnp.kron(jnp.eye(ipp, dtype=conv_wt.dtype),
                     jnp.kron(conv_wt, jnp.eye(8, dtype=conv_wt.dtype)))
    cb_big = jnp.kron(jnp.ones((ipp, 1), conv_b.dtype),
                      jnp.kron(conv_b, jnp.ones((8, 1), conv_b.dtype)))
    fw_bf = bk_fc_w.astype(jnp.bfloat16)

    A_hat = _build_norm_adj(node_num)
    A_bd = jnp.kron(jnp.eye(G, dtype=jnp.float32), A_hat).astype(jnp.bfloat16)
    R = jnp.kron(jnp.eye(G, dtype=jnp.float32),
                 jnp.full((1, node_num), 1.0 / node_num, jnp.float32)
                 ).astype(jnp.bfloat16)
    u2d = batch_u.reshape(GN, Fin)

    Np = _round_up(max(num_classes, 128), 128)
    w2p = jnp.pad(fc_w2, ((0, 0), (0, Np - num_classes)))
    b2p = jnp.pad(fc_b2, ((0, 0), (0, Np - num_classes)))

    n_steps = n_img_steps + 2

    operands = (x48, w_big, cb_big, fw_bf, bk_fc_b,
                u2d, fn_w1, fn_b1, fn_w2, fn_b2,
                A_bd, gcn_w1, gcn_b1, gcn_w2, gcn_b2, gcn_w3, gcn_b3, R,
                fc_w1a, fc_w1b, fc_b1, w2p, b2p)
    vbufs = [pltpu.VMEM(op.shape, op.dtype) for op in operands[1:]]

    flops = (2 * B * HW * C * conv_mid + 2 * B * conv_mid * feat_dim
             + 2 * GN * (Fin * H256 + H256 * ns) + 2 * GN * GN * H1
             + 2 * GN * GN * (H1 + H2 + Fo) + 2 * GN * (H1 * H2 + H2 * Fo)
             + 2 * G * GN * Fo + 2 * B * (F1 * HC + F2 * HC + HC * Np))
    bytes_acc = (B * C * HW * 4 + conv_mid * feat_dim * 2
                 + GN * Fin * 4 + GN * GN * 2 + GN * (H1 + H2) * 2
                 + H2 * Fo * 2 + (F1 + F2) * HC * 2 + HC * Np * 2
                 + B * Np * 4)

    out = pl.pallas_call(
        functools.partial(_fused_kernel, n_img_steps=n_img_steps,
                          num_graphs=G, node_size=node_num,
                          conv_mid=conv_mid, inv_hw=1.0 / float(HW)),
        out_shape=jax.ShapeDtypeStruct((B, Np), jnp.float32),
        grid_spec=pltpu.PrefetchScalarGridSpec(
            num_scalar_prefetch=0,
            grid=(n_steps,),
            in_specs=[pl.BlockSpec(
                (1, rows, lanes),
                lambda sv: (jnp.minimum(sv, n_img_steps - 1), 0, 0))]
            + [pl.BlockSpec(memory_space=pl.ANY)] * (len(operands) - 1),
            out_specs=pl.BlockSpec((B, Np), lambda sv: (0, 0)),
            scratch_shapes=[pltpu.VMEM((B, feat_dim), jnp.bfloat16),
                            pltpu.VMEM((G, Fo), jnp.bfloat16),
                            vbufs,
                            pltpu.SemaphoreType.DMA((len(operands) - 1,))],
        ),
        compiler_params=pltpu.CompilerParams(
            dimension_semantics=("arbitrary",),
            vmem_limit_bytes=100 * 1024 * 1024),
        cost_estimate=pl.CostEstimate(flops=flops, transcendentals=0,
                                      bytes_accessed=bytes_acc),
    )(*operands)
    return out[:, :num_classes]


# speedup vs baseline: 1.0815x; 1.0016x over previous
import functools

import jax
import jax.numpy as jnp
from jax import lax
from jax.experimental import pallas as pl
from jax.experimental.pallas import tpu as pltpu


def _round_up(x, m):
    return ((x + m - 1) // m) * m


def _fused_kernel(x_ref,
                  wbig_ref, cbig_ref, fw_ref, fb_ref,
                  u_ref, fnw1_ref, fnb1_ref, fnw2_ref, fnb2_ref,
                  a_ref, w1_ref, b1_ref, w2_ref, b2_ref, w3_ref, b3_ref,
                  r_ref, w1a_ref, w1b_ref, cb1_ref, cw2_ref, cb2_ref,
                  o_ref,
                  feat_sc, fg_sc, bufs, sems,
                  *, n_img_steps, num_graphs, node_size, conv_mid, inv_hw):
    s = pl.program_id(0)
    gi = n_img_steps

    hbm_refs = (wbig_ref, cbig_ref, fw_ref, fb_ref,
                u_ref, fnw1_ref, fnb1_ref, fnw2_ref, fnb2_ref,
                a_ref, w1_ref, b1_ref, w2_ref, b2_ref, w3_ref, b3_ref,
                r_ref, w1a_ref, w1b_ref, cb1_ref, cw2_ref, cb2_ref)
    n_bk = 4

    @pl.when(s == 0)
    def _prologue():
        feat_sc[...] = jnp.zeros_like(feat_sc)
        for i, ref in enumerate(hbm_refs):
            pltpu.make_async_copy(ref, bufs[i], sems.at[i]).start()
        for i in range(n_bk):
            pltpu.make_async_copy(hbm_refs[i], bufs[i], sems.at[i]).wait()

    @pl.when(s == gi)
    def _wait_tail():
        for i in range(n_bk, len(hbm_refs)):
            pltpu.make_async_copy(hbm_refs[i], bufs[i], sems.at[i]).wait()

    (wbig, cbig, fw, fb, u_b, fnw1, fnb1, fnw2, fnb2,
     a_b, w1_b, b1_b, w2_b, b2_b, w3_b, b3_b, r_b,
     w1a, w1b, cb1, cw2, cb2) = bufs

    @pl.when(s < n_img_steps)
    def _backbone():
        x = x_ref[0].astype(jnp.bfloat16)
        h = jnp.dot(wbig[...], x, preferred_element_type=jnp.float32)
        h = jnp.maximum(h + cbig[...], 0.0)
        L = h.shape[1]
        acc = h[:, 0:128]
        for k in range(1, L // 128):
            acc = acc + h[:, 128 * k:128 * (k + 1)]
        ps = jnp.sum(acc, axis=1)
        pooled = jnp.sum(ps.reshape(2, conv_mid, 8), axis=2) * inv_hw
        feat = jnp.dot(pooled.astype(jnp.bfloat16), fw[...],
                       preferred_element_type=jnp.float32) + fb[...]
        featb = jnp.maximum(feat, 0.0).astype(jnp.bfloat16)
        rows_i = lax.broadcasted_iota(jnp.int32, (feat_sc.shape[0], 1), 0)
        m0 = (rows_i == 2 * s).astype(jnp.bfloat16)
        m1 = (rows_i == 2 * s + 1).astype(jnp.bfloat16)
        feat_sc[...] += m0 * featb[0:1, :] + m1 * featb[1:2, :]

    @pl.when(s == gi)
    def _gcn():
        GN = num_graphs * node_size
        u = u_b[...].astype(jnp.bfloat16)
        t = jnp.dot(u, fnw1[...],
                    preferred_element_type=jnp.float32) + fnb1[...]
        t = jnp.maximum(t, 0.0).astype(jnp.bfloat16)
        h0 = jnp.dot(t, fnw2[...],
                     preferred_element_type=jnp.float32) + fnb2[...]

        A = a_b[...]
        h0b = h0.astype(jnp.bfloat16)
        tiled = jnp.concatenate([h0b] * num_graphs, axis=1)
        row_g = lax.broadcasted_iota(jnp.int32, (GN, GN), 0) // node_size
        col_g = lax.broadcasted_iota(jnp.int32, (GN, GN), 1) // node_size
        ht = jnp.where(row_g == col_g, tiled, jnp.bfloat16(0))
        z = jnp.dot(ht, w1_b[...], preferred_element_type=jnp.float32)
        z = jnp.dot(A, z.astype(jnp.bfloat16),
                    preferred_element_type=jnp.float32) + b1_b[...]
        h1 = jnp.maximum(z, 0.0).astype(jnp.bfloat16)

        z = jnp.dot(h1, w2_b[...], preferred_element_type=jnp.float32)
        z = jnp.dot(A, z.astype(jnp.bfloat16),
                    preferred_element_type=jnp.float32) + b2_b[...]
        h2 = jnp.maximum(z, 0.0).astype(jnp.bfloat16)

        z = jnp.dot(h2, w3_b[...], preferred_element_type=jnp.float32)
        z = jnp.dot(A, z.astype(jnp.bfloat16),
                    preferred_element_type=jnp.float32) + b3_b[...]
        fg = jnp.dot(r_b[...], z.astype(jnp.bfloat16),
                     preferred_element_type=jnp.float32)
        fg_sc[...] = jnp.maximum(fg, 0.0).astype(jnp.bfloat16)

    @pl.when(s == gi + 1)
    def _classifier():
        h = jnp.dot(feat_sc[...], w1a[...],
                    preferred_element_type=jnp.float32)
        h = h + jnp.dot(fg_sc[...], w1b[...],
                        preferred_element_type=jnp.float32)
        h = jnp.maximum(h + cb1[...], 0.0).astype(jnp.bfloat16)
        o_ref[...] = jnp.dot(h, cw2[...],
                             preferred_element_type=jnp.float32) + cb2[...]


def _build_norm_adj(node_num):
    idx = jnp.arange(node_num)
    A = jnp.zeros((node_num, node_num), jnp.float32)
    A = A.at[idx, (idx + 1) % node_num].set(1.0)
    A = A.at[(idx + 1) % node_num, idx].set(1.0)
    A = A + jnp.eye(node_num, dtype=jnp.float32)
    dinv = 1.0 / jnp.sqrt(A.sum(axis=1))
    return A * dinv[:, None] * dinv[None, :]


def kernel(batch_img, batch_u, conv_wt, conv_b, bk_fc_w, bk_fc_b,
           fn_w1, fn_b1, fn_w2, fn_b2, gcn_w1, gcn_b1, gcn_w2, gcn_b2,
           gcn_w3, gcn_b3, fc_w1a, fc_b1, fc_w1b, fc_w2, fc_b2):
    B, C, H, W = batch_img.shape
    HW = H * W
    G, node_num, Fin = batch_u.shape
    GN = G * node_num
    conv_mid = conv_wt.shape[0]
    feat_dim = bk_fc_w.shape[1]
    H256 = fn_w1.shape[1]
    ns = fn_w2.shape[1]
    H1 = gcn_w1.shape[1]
    H2 = gcn_w2.shape[1]
    Fo = gcn_w3.shape[1]
    F1 = fc_w1a.shape[0]
    F2 = fc_w1b.shape[0]
    HC = fc_w1a.shape[1]
    num_classes = fc_w2.shape[1]

    ipp = 2
    rows = 24 * ipp
    lanes = C * HW // 24
    n_img_steps = B // ipp
    x48 = batch_img.reshape(n_img_steps, rows, lanes)
    w_big = jnp.kron(jnp.eye(ipp, dtype=conv_wt.dtype),
                     jnp.kron(conv_wt, jnp.eye(8, dtype=conv_wt.dtype)))
    cb_big = jnp.kron(jnp.ones((ipp, 1), conv_b.dtype),
                      jnp.kron(conv_b, jnp.ones((8, 1), conv_b.dtype)))
    fw_bf = bk_fc_w.astype(jnp.bfloat16)

    A_hat = _build_norm_adj(node_num)
    A_bd = jnp.kron(jnp.eye(G, dtype=jnp.float32), A_hat).astype(jnp.bfloat16)
    R = jnp.kron(jnp.eye(G, dtype=jnp.float32),
                 jnp.full((1, node_num), 1.0 / node_num, jnp.float32)
                 ).astype(jnp.bfloat16)
    u2d = batch_u.reshape(GN, Fin)

    Np = _round_up(max(num_classes, 128), 128)
    w2p = jnp.pad(fc_w2, ((0, 0), (0, Np - num_classes)))
    b2p = jnp.pad(fc_b2, ((0, 0), (0, Np - num_classes)))

    n_steps = n_img_steps + 2

    operands = (x48, w_big, cb_big, fw_bf, bk_fc_b,
                u2d, fn_w1, fn_b1, fn_w2, fn_b2,
                A_bd, gcn_w1, gcn_b1, gcn_w2, gcn_b2, gcn_w3, gcn_b3, R,
                fc_w1a, fc_w1b, fc_b1, w2p, b2p)
    vbufs = [pltpu.VMEM(op.shape, op.dtype) for op in operands[1:]]

    flops = (2 * B * HW * C * conv_mid + 2 * B * conv_mid * feat_dim
             + 2 * GN * (Fin * H256 + H256 * ns) + 2 * GN * GN * H1
             + 2 * GN * GN * (H1 + H2 + Fo) + 2 * GN * (H1 * H2 + H2 * Fo)
             + 2 * G * GN * Fo + 2 * B * (F1 * HC + F2 * HC + HC * Np))
    bytes_acc = (B * C * HW * 4 + conv_mid * feat_dim * 2
                 + GN * Fin * 4 + GN * GN * 2 + GN * (H1 + H2) * 2
                 + H2 * Fo * 2 + (F1 + F2) * HC * 2 + HC * Np * 2
                 + B * Np * 4)

    out = pl.pallas_call(
        functools.partial(_fused_kernel, n_img_steps=n_img_steps,
                          num_graphs=G, node_size=node_num,
                          conv_mid=conv_mid, inv_hw=1.0 / float(HW)),
        out_shape=jax.ShapeDtypeStruct((B, Np), jnp.float32),
        grid_spec=pltpu.PrefetchScalarGridSpec(
            num_scalar_prefetch=0,
            grid=(n_steps,),
            in_specs=[pl.BlockSpec(
                (1, rows, lanes),
                lambda sv: (jnp.minimum(sv, n_img_steps - 1), 0, 0))]
            + [pl.BlockSpec(memory_space=pl.ANY)] * (len(operands) - 1),
            out_specs=pl.BlockSpec((B, Np), lambda sv: (0, 0)),
            scratch_shapes=[pltpu.VMEM((B, feat_dim), jnp.bfloat16),
                            pltpu.VMEM((G, Fo), jnp.bfloat16),
                            vbufs,
                            pltpu.SemaphoreType.DMA((len(operands) - 1,))],
        ),
        compiler_params=pltpu.CompilerParams(
            dimension_semantics=("arbitrary",),
            vmem_limit_bytes=100 * 1024 * 1024),
        cost_estimate=pl.CostEstimate(flops=flops, transcendentals=0,
                                      bytes_accessed=bytes_acc),
    )(*operands)
    return out[:, :num_classes]


# speedup vs baseline: 1.0932x; 1.0108x over previous
import functools

import jax
import jax.numpy as jnp
from jax import lax
from jax.experimental import pallas as pl
from jax.experimental.pallas import tpu as pltpu

_NSLOT = 3


def _round_up(x, m):
    return ((x + m - 1) // m) * m


def _fused_kernel(x_ref,
                  wbig_ref, cbig_ref, fw_ref, fb_ref,
                  u_ref, fnw1_ref, fnb1_ref, fnw2_ref, fnb2_ref,
                  a_ref, w1_ref, b1_ref, w2_ref, b2_ref, w3_ref, b3_ref,
                  r_ref, w1a_ref, w1b_ref, cb1_ref, cw2_ref, cb2_ref,
                  o_ref,
                  feat_sc, fg_sc, bufs, sems, xbuf, xsems,
                  *, n_img_steps, num_graphs, node_size, conv_mid, inv_hw):
    s = pl.program_id(0)
    gi = n_img_steps

    hbm_refs = (wbig_ref, cbig_ref, fw_ref, fb_ref,
                u_ref, fnw1_ref, fnb1_ref, fnw2_ref, fnb2_ref,
                a_ref, w1_ref, b1_ref, w2_ref, b2_ref, w3_ref, b3_ref,
                r_ref, w1a_ref, w1b_ref, cb1_ref, cw2_ref, cb2_ref)
    n_bk = 4
    n_w = len(hbm_refs)

    (wbig, cbig, fw, fb, u_b, fnw1, fnb1, fnw2, fnb2,
     a_b, w1_b, b1_b, w2_b, b2_b, w3_b, b3_b, r_b,
     w1a, w1b, cb1, cw2, cb2) = bufs

    def _x_copy(t):
        slot = lax.rem(t, _NSLOT)
        return pltpu.make_async_copy(x_ref.at[t], xbuf.at[slot],
                                     xsems.at[slot])

    def _w_copy(i):
        return pltpu.make_async_copy(hbm_refs[i], bufs[i], sems.at[i])

    @pl.when(s == 0)
    def _prologue():
        feat_sc[...] = jnp.zeros_like(feat_sc)
        for i in range(n_bk):
            _w_copy(i).start()
        _x_copy(0).start()
        _x_copy(1).start()
        for i in range(n_bk):
            _w_copy(i).wait()

    @pl.when(s < n_img_steps)
    def _backbone():
        _x_copy(s).wait()

        @pl.when(s + 2 < n_img_steps)
        def _prefetch():
            _x_copy(s + 2).start()

        for i in range(n_bk, n_w):
            k = (i - n_bk) // 2

            @pl.when(s == k)
            def _start(i=i):
                _w_copy(i).start()

        x = xbuf[lax.rem(s, _NSLOT)].astype(jnp.bfloat16)
        h = jnp.dot(wbig[...], x, preferred_element_type=jnp.float32)
        h = jnp.maximum(h + cbig[...], 0.0)
        ps = jnp.sum(h, axis=1)
        pooled = jnp.sum(ps.reshape(2, conv_mid, 8), axis=2) * inv_hw
        feat = jnp.dot(pooled.astype(jnp.bfloat16), fw[...],
                       preferred_element_type=jnp.float32) + fb[...]
        featb = jnp.maximum(feat, 0.0).astype(jnp.bfloat16)
        rows_i = lax.broadcasted_iota(jnp.int32, (feat_sc.shape[0], 1), 0)
        m0 = (rows_i == 2 * s).astype(jnp.bfloat16)
        m1 = (rows_i == 2 * s + 1).astype(jnp.bfloat16)
        feat_sc[...] += m0 * featb[0:1, :] + m1 * featb[1:2, :]

    @pl.when(s == gi)
    def _gcn():
        for i in range(n_bk, n_w):
            _w_copy(i).wait()

        GN = num_graphs * node_size
        u = u_b[...].astype(jnp.bfloat16)
        t = jnp.dot(u, fnw1[...],
                    preferred_element_type=jnp.float32) + fnb1[...]
        t = jnp.maximum(t, 0.0).astype(jnp.bfloat16)
        h0 = jnp.dot(t, fnw2[...],
                     preferred_element_type=jnp.float32) + fnb2[...]

        A = a_b[...]
        h0b = h0.astype(jnp.bfloat16)
        tiled = jnp.concatenate([h0b] * num_graphs, axis=1)
        row_g = lax.broadcasted_iota(jnp.int32, (GN, GN), 0) // node_size
        col_g = lax.broadcasted_iota(jnp.int32, (GN, GN), 1) // node_size
        ht = jnp.where(row_g == col_g, tiled, jnp.bfloat16(0))
        z = jnp.dot(ht, w1_b[...], preferred_element_type=jnp.float32)
        z = jnp.dot(A, z.astype(jnp.bfloat16),
                    preferred_element_type=jnp.float32) + b1_b[...]
        h1 = jnp.maximum(z, 0.0).astype(jnp.bfloat16)

        z = jnp.dot(h1, w2_b[...], preferred_element_type=jnp.float32)
        z = jnp.dot(A, z.astype(jnp.bfloat16),
                    preferred_element_type=jnp.float32) + b2_b[...]
        h2 = jnp.maximum(z, 0.0).astype(jnp.bfloat16)

        z = jnp.dot(h2, w3_b[...], preferred_element_type=jnp.float32)
        z = jnp.dot(A, z.astype(jnp.bfloat16),
                    preferred_element_type=jnp.float32) + b3_b[...]
        fg = jnp.dot(r_b[...], z.astype(jnp.bfloat16),
                     preferred_element_type=jnp.float32)
        fg_sc[...] = jnp.maximum(fg, 0.0).astype(jnp.bfloat16)

    @pl.when(s == gi + 1)
    def _classifier():
        h = jnp.dot(feat_sc[...], w1a[...],
                    preferred_element_type=jnp.float32)
        h = h + jnp.dot(fg_sc[...], w1b[...],
                        preferred_element_type=jnp.float32)
        h = jnp.maximum(h + cb1[...], 0.0).astype(jnp.bfloat16)
        o_ref[...] = jnp.dot(h, cw2[...],
                             preferred_element_type=jnp.float32) + cb2[...]


def _build_norm_adj(node_num):
    idx = jnp.arange(node_num)
    A = jnp.zeros((node_num, node_num), jnp.float32)
    A = A.at[idx, (idx + 1) % node_num].set(1.0)
    A = A.at[(idx + 1) % node_num, idx].set(1.0)
    A = A + jnp.eye(node_num, dtype=jnp.float32)
    dinv = 1.0 / jnp.sqrt(A.sum(axis=1))
    return A * dinv[:, None] * dinv[None, :]


def kernel(batch_img, batch_u, conv_wt, conv_b, bk_fc_w, bk_fc_b,
           fn_w1, fn_b1, fn_w2, fn_b2, gcn_w1, gcn_b1, gcn_w2, gcn_b2,
           gcn_w3, gcn_b3, fc_w1a, fc_b1, fc_w1b, fc_w2, fc_b2):
    B, C, H, W = batch_img.shape
    HW = H * W
    G, node_num, Fin = batch_u.shape
    GN = G * node_num
    conv_mid = conv_wt.shape[0]
    feat_dim = bk_fc_w.shape[1]
    H256 = fn_w1.shape[1]
    ns = fn_w2.shape[1]
    H1 = gcn_w1.shape[1]
    H2 = gcn_w2.shape[1]
    Fo = gcn_w3.shape[1]
    F1 = fc_w1a.shape[0]
    F2 = fc_w1b.shape[0]
    HC = fc_w1a.shape[1]
    num_classes = fc_w2.shape[1]

    ipp = 2
    rows = 24 * ipp
    lanes = C * HW // 24
    n_img_steps = B // ipp
    x48 = batch_img.reshape(n_img_steps, rows, lanes)
    w_big = jnp.kron(jnp.eye(ipp, dtype=conv_wt.dtype),
                     jnp.kron(conv_wt, jnp.eye(8, dtype=conv_wt.dtype)))
    cb_big = jnp.kron(jnp.ones((ipp, 1), conv_b.dtype),
                      jnp.kron(conv_b, jnp.ones((8, 1), conv_b.dtype)))
    fw_bf = bk_fc_w.astype(jnp.bfloat16)

    A_hat = _build_norm_adj(node_num)
    A_bd = jnp.kron(jnp.eye(G, dtype=jnp.float32), A_hat).astype(jnp.bfloat16)
    R = jnp.kron(jnp.eye(G, dtype=jnp.float32),
                 jnp.full((1, node_num), 1.0 / node_num, jnp.float32)
                 ).astype(jnp.bfloat16)
    u2d = batch_u.reshape(GN, Fin)

    Np = _round_up(max(num_classes, 128), 128)
    w2p = jnp.pad(fc_w2, ((0, 0), (0, Np - num_classes)))
    b2p = jnp.pad(fc_b2, ((0, 0), (0, Np - num_classes)))

    n_steps = n_img_steps + 2

    operands = (x48, w_big, cb_big, fw_bf, bk_fc_b,
                u2d, fn_w1, fn_b1, fn_w2, fn_b2,
                A_bd, gcn_w1, gcn_b1, gcn_w2, gcn_b2, gcn_w3, gcn_b3, R,
                fc_w1a, fc_w1b, fc_b1, w2p, b2p)
    vbufs = [pltpu.VMEM(op.shape, op.dtype) for op in operands[1:]]

    flops = (2 * B * HW * C * conv_mid + 2 * B * conv_mid * feat_dim
             + 2 * GN * (Fin * H256 + H256 * ns) + 2 * GN * GN * H1
             + 2 * GN * GN * (H1 + H2 + Fo) + 2 * GN * (H1 * H2 + H2 * Fo)
             + 2 * G * GN * Fo + 2 * B * (F1 * HC + F2 * HC + HC * Np))
    bytes_acc = (B * C * HW * 4 + conv_mid * feat_dim * 2
                 + GN * Fin * 4 + GN * GN * 2 + GN * (H1 + H2) * 2
                 + H2 * Fo * 2 + (F1 + F2) * HC * 2 + HC * Np * 2
                 + B * Np * 4)

    out = pl.pallas_call(
        functools.partial(_fused_kernel, n_img_steps=n_img_steps,
                          num_graphs=G, node_size=node_num,
                          conv_mid=conv_mid, inv_hw=1.0 / float(HW)),
        out_shape=jax.ShapeDtypeStruct((B, Np), jnp.float32),
        grid_spec=pltpu.PrefetchScalarGridSpec(
            num_scalar_prefetch=0,
            grid=(n_steps,),
            in_specs=[pl.BlockSpec(memory_space=pl.ANY)] * len(operands),
            out_specs=pl.BlockSpec((B, Np), lambda sv: (0, 0)),
            scratch_shapes=[pltpu.VMEM((B, feat_dim), jnp.bfloat16),
                            pltpu.VMEM((G, Fo), jnp.bfloat16),
                            vbufs,
                            pltpu.SemaphoreType.DMA((len(operands) - 1,)),
                            pltpu.VMEM((_NSLOT, rows, lanes), jnp.float32),
                            pltpu.SemaphoreType.DMA((_NSLOT,))],
        ),
        compiler_params=pltpu.CompilerParams(
            dimension_semantics=("arbitrary",),
            vmem_limit_bytes=100 * 1024 * 1024),
        cost_estimate=pl.CostEstimate(flops=flops, transcendentals=0,
                                      bytes_accessed=bytes_acc),
    )(*operands)
    return out[:, :num_classes]
